# baseline (speedup 1.0000x reference)
import functools

import jax
import jax.numpy as jnp
from jax import lax
from jax.experimental import pallas as pl
from jax.experimental.pallas import tpu as pltpu
from jax.experimental.pallas import tpu_sc as plsc

_NC = 1
_NS = 16
_NW = _NC * _NS


def _sc_scatter_body(new_ref, rowidx_ref, out_ref, newbuf, idxbuf,
                     gsem, ssem):
    npairs = new_ref.shape[0]
    w = lax.axis_index("s") * _NC + lax.axis_index("c")
    pairs_w = npairs // _NW
    stages = []
    for j in range(pairs_w):
        bg = w * pairs_w + j
        stages.append(pltpu.make_async_copy(
            new_ref.at[bg], newbuf.at[j], gsem))
        stages.append(pltpu.make_async_copy(
            rowidx_ref.at[bg], idxbuf.at[j], gsem))
    for d in stages:
        d.start()
    for d in stages:
        d.wait()
    scats = [pltpu.make_async_copy(
        newbuf.at[j], out_ref.at[idxbuf.at[j]], ssem)
        for j in range(pairs_w)]
    for d in scats:
        d.start()
    for d in scats:
        d.wait()


def kernel(key, value, k_cache, v_cache, next_token_pos):
    B, G, L, H = k_cache.shape
    num = key.shape[2]
    BG = B * G

    key2 = key.reshape(BG, num, H)
    value2 = value.reshape(BG, num, H)
    pos = jnp.asarray(next_token_pos, jnp.int32)
    rowidx = (jnp.arange(BG, dtype=jnp.int32)[:, None] * L + pos
              + jnp.arange(num, dtype=jnp.int32)[None, :])

    mesh = plsc.VectorSubcoreMesh(core_axis_name="c", subcore_axis_name="s",
                                  num_cores=_NC, num_subcores=_NS)
    sc_scatter = functools.partial(
        pl.kernel,
        out_type=(),
        mesh=mesh,
        scratch_types=[
            pltpu.VMEM((BG // _NW, num, H), k_cache.dtype),
            pltpu.VMEM((BG // _NW, num), jnp.int32),
            pltpu.SemaphoreType.DMA,
            pltpu.SemaphoreType.DMA,
        ],
    )(_sc_scatter_body)

    ko_ref = jax.new_ref(k_cache.reshape(BG * L, H))
    sc_scatter(key2, rowidx, ko_ref)
    vo_ref = jax.new_ref(v_cache.reshape(BG * L, H))
    sc_scatter(value2, rowidx, vo_ref)

    return (ko_ref[...].reshape(B, G, L, H),
            vo_ref[...].reshape(B, G, L, H))


# speedup vs baseline: 1.0139x; 1.0139x over previous
import functools

import jax
import jax.numpy as jnp
from jax import lax
from jax.experimental import pallas as pl
from jax.experimental.pallas import tpu as pltpu
from jax.experimental.pallas import tpu_sc as plsc

_NC = 1
_NS = 16
_NW = _NC * _NS


def _sc_scatter_body(key_ref, value_ref, rowidx_ref, ko_ref, vo_ref,
                     newbuf, idxbuf, gsem, ssem):
    npairs = key_ref.shape[0]
    w = lax.axis_index("s") * _NC + lax.axis_index("c")
    pairs_w = npairs // _NW
    stages = []
    for j in range(pairs_w):
        bg = w * pairs_w + j
        stages.append(pltpu.make_async_copy(
            key_ref.at[bg], newbuf.at[2 * j], gsem))
        stages.append(pltpu.make_async_copy(
            value_ref.at[bg], newbuf.at[2 * j + 1], gsem))
        stages.append(pltpu.make_async_copy(
            rowidx_ref.at[bg], idxbuf.at[j], gsem))
    for d in stages:
        d.start()
    for d in stages:
        d.wait()
    scats = []
    for j in range(pairs_w):
        scats.append(pltpu.make_async_copy(
            newbuf.at[2 * j], ko_ref.at[idxbuf.at[j]], ssem))
        scats.append(pltpu.make_async_copy(
            newbuf.at[2 * j + 1], vo_ref.at[idxbuf.at[j]], ssem))
    for d in scats:
        d.start()
    for d in scats:
        d.wait()


def kernel(key, value, k_cache, v_cache, next_token_pos):
    B, G, L, H = k_cache.shape
    num = key.shape[2]
    BG = B * G

    key2 = key.reshape(BG, num, H)
    value2 = value.reshape(BG, num, H)
    pos = jnp.asarray(next_token_pos, jnp.int32)
    rowidx = (jnp.arange(BG, dtype=jnp.int32)[:, None] * L + pos
              + jnp.arange(num, dtype=jnp.int32)[None, :])

    ko_ref = jax.new_ref(k_cache.reshape(BG * L, H))
    vo_ref = jax.new_ref(v_cache.reshape(BG * L, H))

    mesh = plsc.VectorSubcoreMesh(core_axis_name="c", subcore_axis_name="s",
                                  num_cores=_NC, num_subcores=_NS)
    sc_scatter = functools.partial(
        pl.kernel,
        out_type=(),
        mesh=mesh,
        scratch_types=[
            pltpu.VMEM((2 * (BG // _NW), num, H), k_cache.dtype),
            pltpu.VMEM((BG // _NW, num), jnp.int32),
            pltpu.SemaphoreType.DMA,
            pltpu.SemaphoreType.DMA,
        ],
    )(_sc_scatter_body)
    sc_scatter(key2, value2, rowidx, ko_ref, vo_ref)

    return (ko_ref[...].reshape(B, G, L, H),
            vo_ref[...].reshape(B, G, L, H))


# speedup vs baseline: 1.0145x; 1.0006x over previous
import functools

import jax
import jax.numpy as jnp
from jax import lax
from jax.experimental import pallas as pl
from jax.experimental.pallas import tpu as pltpu
from jax.experimental.pallas import tpu_sc as plsc

_NC = 1
_NS = 16
_NW = _NC * _NS


def _sc_scatter_body(key_ref, value_ref, rowidx_ref, ko_ref, vo_ref,
                     kbuf, vbuf, idxbuf, gsem, ssem):
    nrows = key_ref.shape[0]
    w = lax.axis_index("s") * _NC + lax.axis_index("c")
    rows_w = nrows // _NW
    sl = pl.ds(w * rows_w, rows_w)
    stages = [
        pltpu.make_async_copy(key_ref.at[sl], kbuf, gsem),
        pltpu.make_async_copy(value_ref.at[sl], vbuf, gsem),
        pltpu.make_async_copy(rowidx_ref.at[sl], idxbuf, gsem),
    ]
    for d in stages:
        d.start()
    for d in stages:
        d.wait()
    scats = [
        pltpu.make_async_copy(kbuf, ko_ref.at[idxbuf], ssem),
        pltpu.make_async_copy(vbuf, vo_ref.at[idxbuf], ssem),
    ]
    for d in scats:
        d.start()
    for d in scats:
        d.wait()


def kernel(key, value, k_cache, v_cache, next_token_pos):
    B, G, L, H = k_cache.shape
    num = key.shape[2]
    BG = B * G
    rows_w = (BG * num) // _NW

    key2 = key.reshape(BG * num, H)
    value2 = value.reshape(BG * num, H)
    pos = jnp.asarray(next_token_pos, jnp.int32)
    rowidx = (jnp.arange(BG, dtype=jnp.int32)[:, None] * L + pos
              + jnp.arange(num, dtype=jnp.int32)[None, :]).reshape(BG * num)

    ko_ref = jax.new_ref(k_cache.reshape(BG * L, H))
    vo_ref = jax.new_ref(v_cache.reshape(BG * L, H))

    mesh = plsc.VectorSubcoreMesh(core_axis_name="c", subcore_axis_name="s",
                                  num_cores=_NC, num_subcores=_NS)
    sc_scatter = functools.partial(
        pl.kernel,
        out_type=(),
        mesh=mesh,
        scratch_types=[
            pltpu.VMEM((rows_w, H), k_cache.dtype),
            pltpu.VMEM((rows_w, H), v_cache.dtype),
            pltpu.VMEM((rows_w,), jnp.int32),
            pltpu.SemaphoreType.DMA,
            pltpu.SemaphoreType.DMA,
        ],
    )(_sc_scatter_body)
    sc_scatter(key2, value2, rowidx, ko_ref, vo_ref)

    return (ko_ref[...].reshape(B, G, L, H),
            vo_ref[...].reshape(B, G, L, H))
